# baseline (speedup 1.0000x reference)
import jax
import jax.numpy as jnp
from jax import lax
from jax.experimental import pallas as pl
from jax.experimental.pallas import tpu as pltpu
from jax.experimental.pallas import tpu_sc as plsc

N_NODES = 10000
D = 128
N_EDGES = 320000

NC = 2
NS = 16
NW = NC * NS
E_PER_W = N_EDGES // NW
CHUNK = 32
N_GROUPS = CHUNK // 16

N_FULL = E_PER_W // CHUNK
N_CHUNKS = N_FULL + 1 + (N_FULL + 1) % 2


def _dot_chunk(rows_buf, out_v, out_base):
    u_rows = [(lax.iota(jnp.int32, 16) + g * 16) * 2 for g in range(N_GROUPS)]
    v_rows = [r + 1 for r in u_rows]

    def body(d, accs):
        col = jnp.full((16,), d, dtype=jnp.int32)
        new = []
        for g in range(N_GROUPS):
            gu = plsc.load_gather(rows_buf, [u_rows[g], col])
            gv = plsc.load_gather(rows_buf, [v_rows[g], col])
            new.append(accs[g] + gu * gv)
        return tuple(new)

    accs = lax.fori_loop(0, D, body,
                         tuple(jnp.zeros((16,), jnp.float32)
                               for _ in range(N_GROUPS)))
    for g in range(N_GROUPS):
        out_v[pl.ds(out_base + g * 16, 16)] = accs[g]


def _sc_kernel(h_hbm, w_hbm, out_hbm,
               h_sp, w_idx, b0, b1, out_v, sem0, sem1):
    bufs = [b0, b1]
    sems = [sem0, sem1]

    sid = lax.axis_index("s")
    wid = sid * NC + lax.axis_index("c")
    base = wid * E_PER_W

    @pl.when(sid == 0)
    def _stage_h():
        pltpu.sync_copy(h_hbm, h_sp)

    pltpu.sync_copy(w_hbm.at[pl.ds(2 * base, 2 * E_PER_W)], w_idx)
    plsc.subcore_barrier()

    def chunk_off(c):
        return jnp.minimum(c * CHUNK, E_PER_W - CHUNK)

    def issue(c, slot):
        off = chunk_off(c)
        pltpu.async_copy(h_sp.at[w_idx.at[pl.ds(2 * off, 2 * CHUNK)]],
                         bufs[slot], sems[slot])

    def wait(slot):
        dummy = h_hbm.at[pl.ds(0, 2 * CHUNK)]
        pltpu.make_async_copy(dummy, bufs[slot], sems[slot]).wait()

    issue(0, 0)

    def pair_body(i, carry):
        c0 = 2 * i
        wait(0)
        issue(c0 + 1, 1)
        _dot_chunk(b0, out_v, chunk_off(c0))
        wait(1)

        @pl.when(c0 + 2 < N_CHUNKS)
        def _issue_next():
            issue(c0 + 2, 0)

        _dot_chunk(b1, out_v, chunk_off(c0 + 1))
        return carry

    lax.fori_loop(0, N_CHUNKS // 2, pair_body, 0)

    pltpu.sync_copy(out_v, out_hbm.at[pl.ds(base, E_PER_W)])


@jax.jit
def _run(h, w):
    mesh = plsc.VectorSubcoreMesh(core_axis_name="c", subcore_axis_name="s",
                                  num_cores=NC, num_subcores=NS)
    return pl.kernel(
        _sc_kernel,
        out_type=jax.ShapeDtypeStruct((N_EDGES,), jnp.float32),
        mesh=mesh,
        scratch_types=[
            pltpu.VMEM_SHARED((N_NODES, D), jnp.float32),
            pltpu.VMEM((2 * E_PER_W,), jnp.int32),
            pltpu.VMEM((2 * CHUNK, D), jnp.float32),
            pltpu.VMEM((2 * CHUNK, D), jnp.float32),
            pltpu.VMEM((E_PER_W,), jnp.float32),
            pltpu.SemaphoreType.DMA,
            pltpu.SemaphoreType.DMA,
        ],
        compiler_params=pltpu.CompilerParams(needs_layout_passes=False),
    )(h, w)


def kernel(g, h, u, v):
    w = jnp.stack([u.astype(jnp.int32), v.astype(jnp.int32)],
                  axis=1).reshape(-1)
    return _run(h, w)


# speedup vs baseline: 1.1401x; 1.1401x over previous
import jax
import jax.numpy as jnp
from jax import lax
from jax.experimental import pallas as pl
from jax.experimental.pallas import tpu as pltpu
from jax.experimental.pallas import tpu_sc as plsc

N_NODES = 10000
D = 128
N_EDGES = 320000

NC = 2
NS = 16
NW = NC * NS
E_PER_W = N_EDGES // NW
CHUNK = 32
N_GROUPS = CHUNK // 16

N_FULL = E_PER_W // CHUNK
N_CHUNKS = N_FULL + 1 + (N_FULL + 1) % 2


def _dot_chunk(u_rows, v_rows, out_v, out_base):
    rows = [lax.iota(jnp.int32, 16) + g * 16 for g in range(N_GROUPS)]

    def body(d, accs):
        col = jnp.full((16,), d, dtype=jnp.int32)
        new = []
        for g in range(N_GROUPS):
            gu = plsc.load_gather(u_rows, [rows[g], col])
            gv = plsc.load_gather(v_rows, [rows[g], col])
            new.append(accs[g] + gu * gv)
        return tuple(new)

    accs = lax.fori_loop(0, D, body,
                         tuple(jnp.zeros((16,), jnp.float32)
                               for _ in range(N_GROUPS)))
    for g in range(N_GROUPS):
        out_v[pl.ds(out_base + g * 16, 16)] = accs[g]


def _sc_kernel(h_hbm, u_hbm, v_hbm, out_hbm,
               h_sp, u_idx, v_idx, u_b0, u_b1, v_b0, v_b1, out_v,
               sem0, sem1):
    u_bufs = [u_b0, u_b1]
    v_bufs = [v_b0, v_b1]
    sems = [sem0, sem1]

    sid = lax.axis_index("s")
    wid = sid * NC + lax.axis_index("c")
    base = wid * E_PER_W

    @pl.when(sid == 0)
    def _stage_h():
        pltpu.sync_copy(h_hbm, h_sp)

    pltpu.sync_copy(u_hbm.at[pl.ds(base, E_PER_W)], u_idx)
    pltpu.sync_copy(v_hbm.at[pl.ds(base, E_PER_W)], v_idx)
    plsc.subcore_barrier()

    def chunk_off(c):
        return jnp.minimum(c * CHUNK, E_PER_W - CHUNK)

    def issue(c, slot):
        off = chunk_off(c)
        pltpu.async_copy(h_sp.at[u_idx.at[pl.ds(off, CHUNK)]],
                         u_bufs[slot], sems[slot])
        pltpu.async_copy(h_sp.at[v_idx.at[pl.ds(off, CHUNK)]],
                         v_bufs[slot], sems[slot])

    def wait(slot):
        dummy = h_hbm.at[pl.ds(0, CHUNK)]
        pltpu.make_async_copy(dummy, u_bufs[slot], sems[slot]).wait()
        pltpu.make_async_copy(dummy, v_bufs[slot], sems[slot]).wait()

    issue(0, 0)

    def pair_body(i, carry):
        c0 = 2 * i
        wait(0)
        issue(c0 + 1, 1)
        _dot_chunk(u_b0, v_b0, out_v, chunk_off(c0))
        wait(1)

        @pl.when(c0 + 2 < N_CHUNKS)
        def _issue_next():
            issue(c0 + 2, 0)

        _dot_chunk(u_b1, v_b1, out_v, chunk_off(c0 + 1))
        return carry

    lax.fori_loop(0, N_CHUNKS // 2, pair_body, 0)

    pltpu.sync_copy(out_v, out_hbm.at[pl.ds(base, E_PER_W)])


@jax.jit
def _run(h, u, v):
    mesh = plsc.VectorSubcoreMesh(core_axis_name="c", subcore_axis_name="s",
                                  num_cores=NC, num_subcores=NS)
    return pl.kernel(
        _sc_kernel,
        out_type=jax.ShapeDtypeStruct((N_EDGES,), jnp.float32),
        mesh=mesh,
        scratch_types=[
            pltpu.VMEM_SHARED((N_NODES, D), jnp.float32),
            pltpu.VMEM((E_PER_W,), jnp.int32),
            pltpu.VMEM((E_PER_W,), jnp.int32),
            pltpu.VMEM((CHUNK, D), jnp.float32),
            pltpu.VMEM((CHUNK, D), jnp.float32),
            pltpu.VMEM((CHUNK, D), jnp.float32),
            pltpu.VMEM((CHUNK, D), jnp.float32),
            pltpu.VMEM((E_PER_W,), jnp.float32),
            pltpu.SemaphoreType.DMA,
            pltpu.SemaphoreType.DMA,
        ],
        compiler_params=pltpu.CompilerParams(needs_layout_passes=False),
    )(h, u, v)


def kernel(g, h, u, v):
    return _run(h, u.astype(jnp.int32), v.astype(jnp.int32))


# speedup vs baseline: 1.2405x; 1.0881x over previous
import jax
import jax.numpy as jnp
from jax import lax
from jax.experimental import pallas as pl
from jax.experimental.pallas import tpu as pltpu
from jax.experimental.pallas import tpu_sc as plsc

N_NODES = 10000
D = 128
N_EDGES = 320000

NC = 2
NS = 16
NW = NC * NS
E_PER_W = N_EDGES // NW
CHUNK = 16
N_GROUPS = CHUNK // 16
RING = 5

N_CHUNKS = E_PER_W // CHUNK


def _dot_chunk(u_rows, v_rows, out_v, out_base):
    rows = [lax.iota(jnp.int32, 16) + g * 16 for g in range(N_GROUPS)]

    def body(d, accs):
        col = jnp.full((16,), d, dtype=jnp.int32)
        new = []
        for g in range(N_GROUPS):
            gu = plsc.load_gather(u_rows, [rows[g], col])
            gv = plsc.load_gather(v_rows, [rows[g], col])
            new.append(accs[g] + gu * gv)
        return tuple(new)

    accs = lax.fori_loop(0, D, body,
                         tuple(jnp.zeros((16,), jnp.float32)
                               for _ in range(N_GROUPS)))
    for g in range(N_GROUPS):
        out_v[pl.ds(out_base + g * 16, 16)] = accs[g]


def _sc_kernel(h_hbm, u_hbm, v_hbm, out_hbm,
               h_sp, u_idx, v_idx,
               u_b0, u_b1, u_b2, u_b3, u_b4, v_b0, v_b1, v_b2, v_b3, v_b4,
               out_v, sem0, sem1, sem2, sem3, sem4):
    u_bufs = [u_b0, u_b1, u_b2, u_b3, u_b4]
    v_bufs = [v_b0, v_b1, v_b2, v_b3, v_b4]
    sems = [sem0, sem1, sem2, sem3, sem4]

    sid = lax.axis_index("s")
    wid = sid * NC + lax.axis_index("c")
    base = wid * E_PER_W

    @pl.when(sid == 0)
    def _stage_h():
        pltpu.sync_copy(h_hbm, h_sp)

    pltpu.sync_copy(u_hbm.at[pl.ds(base, E_PER_W)], u_idx)
    pltpu.sync_copy(v_hbm.at[pl.ds(base, E_PER_W)], v_idx)
    plsc.subcore_barrier()

    def issue(c, slot):
        off = c * CHUNK
        pltpu.async_copy(h_sp.at[u_idx.at[pl.ds(off, CHUNK)]],
                         u_bufs[slot], sems[slot])
        pltpu.async_copy(h_sp.at[v_idx.at[pl.ds(off, CHUNK)]],
                         v_bufs[slot], sems[slot])

    def wait(slot):
        dummy = h_hbm.at[pl.ds(0, CHUNK)]
        pltpu.make_async_copy(dummy, u_bufs[slot], sems[slot]).wait()
        pltpu.make_async_copy(dummy, v_bufs[slot], sems[slot]).wait()

    for j in range(RING - 1):
        issue(j, j)

    def ring_body(i, carry):
        c0 = i * RING
        for j in range(RING):
            c = c0 + j
            nxt = c + RING - 1

            @pl.when(nxt < N_CHUNKS)
            def _issue_next():
                issue(nxt, (j + RING - 1) % RING)

            wait(j)
            _dot_chunk(u_bufs[j], v_bufs[j], out_v, c * CHUNK)
        return carry

    lax.fori_loop(0, N_CHUNKS // RING, ring_body, 0)

    pltpu.sync_copy(out_v, out_hbm.at[pl.ds(base, E_PER_W)])


@jax.jit
def _run(h, u, v):
    mesh = plsc.VectorSubcoreMesh(core_axis_name="c", subcore_axis_name="s",
                                  num_cores=NC, num_subcores=NS)
    return pl.kernel(
        _sc_kernel,
        out_type=jax.ShapeDtypeStruct((N_EDGES,), jnp.float32),
        mesh=mesh,
        scratch_types=[
            pltpu.VMEM_SHARED((N_NODES, D), jnp.float32),
            pltpu.VMEM((E_PER_W,), jnp.int32),
            pltpu.VMEM((E_PER_W,), jnp.int32),
            pltpu.VMEM((CHUNK, D), jnp.float32),
            pltpu.VMEM((CHUNK, D), jnp.float32),
            pltpu.VMEM((CHUNK, D), jnp.float32),
            pltpu.VMEM((CHUNK, D), jnp.float32),
            pltpu.VMEM((CHUNK, D), jnp.float32),
            pltpu.VMEM((CHUNK, D), jnp.float32),
            pltpu.VMEM((CHUNK, D), jnp.float32),
            pltpu.VMEM((CHUNK, D), jnp.float32),
            pltpu.VMEM((CHUNK, D), jnp.float32),
            pltpu.VMEM((CHUNK, D), jnp.float32),
            pltpu.VMEM((E_PER_W,), jnp.float32),
            pltpu.SemaphoreType.DMA,
            pltpu.SemaphoreType.DMA,
            pltpu.SemaphoreType.DMA,
            pltpu.SemaphoreType.DMA,
            pltpu.SemaphoreType.DMA,
        ],
        compiler_params=pltpu.CompilerParams(needs_layout_passes=False),
    )(h, u, v)


def kernel(g, h, u, v):
    return _run(h, u.astype(jnp.int32), v.astype(jnp.int32))
